# Initial kernel scaffold; baseline (speedup 1.0000x reference)
import jax
import jax.numpy as jnp
from jax.experimental import pallas as pl


def kernel(indices, table):
    raise NotImplementedError("write your pallas kernel here")


# speedup vs baseline: 3.0785x; 3.0785x over previous
import functools

import jax
import jax.numpy as jnp
import numpy as np
from jax import lax
from jax.experimental import pallas as pl
from jax.experimental.pallas import tpu as pltpu
from jax.experimental.pallas import tpu_sc as plsc

B = 4096
L = 200
D = 64
NC = 2
NS = 16
NW = NC * NS
SEQ_PER_W = B // NW
CH = 40
NCH = L // CH


def _pos_encoding():
    position = np.arange(0, L, dtype=np.float32)[:, None]
    div_term = np.exp(np.arange(0, D, 2, dtype=np.float32) * (-np.log(10000.0) / D))
    pe = np.zeros((L, D), dtype=np.float32)
    pe[:, 0::2] = np.sin(position * div_term)
    pe[:, 1::2] = np.cos(position * div_term)
    return jnp.asarray(pe)


_MESH = plsc.VectorSubcoreMesh(core_axis_name="c", subcore_axis_name="s")


@functools.partial(
    pl.kernel,
    out_type=jax.ShapeDtypeStruct((B * L, D), jnp.float32),
    mesh=_MESH,
    scratch_types=[
        pltpu.VMEM((NCH, CH), jnp.int32),
        pltpu.VMEM((L, D), jnp.float32),
        pltpu.VMEM((L, D), jnp.float32),
        pltpu.SemaphoreType.DMA,
    ],
    compiler_params=pltpu.CompilerParams(use_tc_tiling_on_sc=False),
)
def _embed(idx_hbm, pe_hbm, table_hbm, out_hbm, idx_v, pe_v, rows_v, sem):
    wid = lax.axis_index("s") * NC + lax.axis_index("c")
    pltpu.sync_copy(pe_hbm, pe_v)

    def seq_body(i, carry):
        seq = wid * SEQ_PER_W + i
        pltpu.sync_copy(idx_hbm.at[seq], idx_v)
        copies = [
            pltpu.async_copy(
                table_hbm.at[idx_v.at[c]], rows_v.at[pl.ds(c * CH, CH)], sem
            )
            for c in range(NCH)
        ]
        for cp in copies:
            cp.wait()

        def row_body(r, rcarry):
            for j in range(D // 16):
                sl = (r, pl.ds(j * 16, 16))
                rows_v[sl] = rows_v[sl] + pe_v[sl]
            return rcarry

        lax.fori_loop(0, L, row_body, 0)
        pltpu.sync_copy(rows_v, out_hbm.at[pl.ds(seq * L, L)])
        return carry

    lax.fori_loop(0, SEQ_PER_W, seq_body, 0)


def kernel(indices, table):
    idx2 = indices.reshape(B, NCH, CH).astype(jnp.int32)
    out = _embed(idx2, _pos_encoding(), table)
    return out.reshape(B, L, D)


# speedup vs baseline: 3.5973x; 1.1685x over previous
import functools

import jax
import jax.numpy as jnp
import numpy as np
from jax import lax
from jax.experimental import pallas as pl
from jax.experimental.pallas import tpu as pltpu
from jax.experimental.pallas import tpu_sc as plsc

B = 4096
L = 200
D = 64
NC = 2
NS = 16
NW = NC * NS
SEQ_PER_W = B // NW
CH = 100
NCH = L // CH


def _pos_encoding():
    position = np.arange(0, L, dtype=np.float32)[:, None]
    div_term = np.exp(np.arange(0, D, 2, dtype=np.float32) * (-np.log(10000.0) / D))
    pe = np.zeros((L, D), dtype=np.float32)
    pe[:, 0::2] = np.sin(position * div_term)
    pe[:, 1::2] = np.cos(position * div_term)
    return jnp.asarray(pe)


_MESH = plsc.VectorSubcoreMesh(core_axis_name="c", subcore_axis_name="s")


@functools.partial(
    pl.kernel,
    out_type=jax.ShapeDtypeStruct((B * L, D), jnp.float32),
    mesh=_MESH,
    scratch_types=[
        pltpu.VMEM((2, NCH, CH), jnp.int32),
        pltpu.VMEM((L, D), jnp.float32),
        pltpu.VMEM((2, L, D), jnp.float32),
        pltpu.SemaphoreType.DMA,
        pltpu.SemaphoreType.DMA,
        pltpu.SemaphoreType.DMA,
        pltpu.SemaphoreType.DMA,
    ],
    compiler_params=pltpu.CompilerParams(use_tc_tiling_on_sc=False),
)
def _embed(idx_hbm, pe_hbm, table_hbm, out_hbm,
           idx_v, pe_v, rows_v, semg0, semg1, semo0, semo1):
    wid = lax.axis_index("s") * NC + lax.axis_index("c")
    base = wid * SEQ_PER_W
    semg = (semg0, semg1)
    semo = (semo0, semo1)
    pltpu.sync_copy(pe_hbm, pe_v)

    def fire_gather(seq, b):
        pltpu.sync_copy(idx_hbm.at[seq], idx_v.at[b])
        for c in range(NCH):
            pltpu.async_copy(
                table_hbm.at[idx_v.at[b].at[c]],
                rows_v.at[b].at[pl.ds(c * CH, CH)],
                semg[b],
            )

    def wait_gather(b):
        for c in range(NCH):
            pltpu.make_async_copy(
                table_hbm.at[idx_v.at[b].at[c]],
                rows_v.at[b].at[pl.ds(c * CH, CH)],
                semg[b],
            ).wait()

    def fire_store(seq, b):
        pltpu.async_copy(rows_v.at[b], out_hbm.at[pl.ds(seq * L, L)], semo[b])

    def wait_store(b):
        pltpu.make_async_copy(
            rows_v.at[b], out_hbm.at[pl.ds(base * L, L)], semo[b]
        ).wait()

    fire_gather(base, 0)

    def pair_body(g, carry):
        for b in range(2):
            s = 2 * g + b
            nb = 1 - b

            @pl.when(s + 1 < SEQ_PER_W)
            def _():
                @pl.when(s >= 1)
                def _():
                    wait_store(nb)

                fire_gather(base + s + 1, nb)

            wait_gather(b)
            rv = rows_v.at[b]

            def row_body(r, rc):
                for rr in range(2):
                    for j in range(D // 16):
                        sl = (2 * r + rr, pl.ds(j * 16, 16))
                        rv[sl] = rv[sl] + pe_v[sl]
                return rc

            lax.fori_loop(0, L // 2, row_body, 0)
            fire_store(base + s, b)
        return carry

    lax.fori_loop(0, SEQ_PER_W // 2, pair_body, 0)
    wait_store(0)
    wait_store(1)


def kernel(indices, table):
    idx2 = indices.reshape(B, NCH, CH).astype(jnp.int32)
    out = _embed(idx2, _pos_encoding(), table)
    return out.reshape(B, L, D)
